# baseline (speedup 1.0000x reference)
import functools

import jax
import jax.numpy as jnp
from jax import lax
from jax.experimental import pallas as pl
from jax.experimental.pallas import tpu as pltpu
from jax.experimental.pallas import tpu_sc as plsc

N = 3200
E = 12800
G = 128
K = 32

NC = 2
NS = 16
NW = NC * NS
EPW = E // NW
CHUNK = 80
NCHUNK = EPW // CHUNK
NPT = N // NS
CNTCOL = 64

_MESH = plsc.VectorSubcoreMesh(core_axis_name="c", subcore_axis_name="s")


def _sc_gather(D):
    @functools.partial(
        pl.kernel,
        out_type=jax.ShapeDtypeStruct((E, D), jnp.float32),
        mesh=_MESH,
        scratch_types=[
            pltpu.VMEM((NCHUNK, CHUNK), jnp.int32),
            pltpu.VMEM((EPW, D), jnp.float32),
            pltpu.SemaphoreType.DMA,
        ],
    )
    def k(table_hbm, idx_hbm, out_hbm, idx_v, rows_v, sem):
        wid = lax.axis_index("s") * NC + lax.axis_index("c")
        pltpu.sync_copy(idx_hbm.at[wid], idx_v)
        copies = [
            pltpu.async_copy(
                table_hbm.at[idx_v.at[j]],
                rows_v.at[pl.ds(j * CHUNK, CHUNK)],
                sem,
            )
            for j in range(NCHUNK)
        ]
        for c in copies:
            c.wait()
        pltpu.sync_copy(rows_v, out_hbm.at[pl.ds(wid * EPW, EPW)])

    return k


def _sc_scatter_add(C):
    @functools.partial(
        pl.kernel,
        out_type=pltpu.HBM((NC, N, C), jnp.float32),
        mesh=_MESH,
        scratch_types=[
            pltpu.VMEM((NCHUNK, CHUNK), jnp.int32),
            pltpu.VMEM((EPW, C), jnp.float32),
            pltpu.VMEM_SHARED((N, C), jnp.float32),
        ],
    )
    def k(rows_hbm, idx_hbm, zeros_hbm, out_hbm, idx_v, rows_v, acc_sh):
        cid = lax.axis_index("c")
        sid = lax.axis_index("s")
        wid = sid * NC + cid
        pltpu.sync_copy(zeros_hbm.at[pl.ds(sid * NPT, NPT)],
                        acc_sh.at[pl.ds(sid * NPT, NPT)])
        pltpu.sync_copy(idx_hbm.at[wid], idx_v)
        pltpu.sync_copy(rows_hbm.at[pl.ds(wid * EPW, EPW)], rows_v)
        plsc.subcore_barrier()
        for j in range(NCHUNK):
            pltpu.sync_copy(rows_v.at[pl.ds(j * CHUNK, CHUNK)],
                            acc_sh.at[idx_v.at[j]], add=True)
        plsc.subcore_barrier()
        pltpu.sync_copy(acc_sh.at[pl.ds(sid * NPT, NPT)],
                        out_hbm.at[cid, pl.ds(sid * NPT, NPT)])

    return k


def _sc_scatter_add2(C):
    @functools.partial(
        pl.kernel,
        out_type=pltpu.HBM((NC, 2, N, C), jnp.float32),
        mesh=_MESH,
        scratch_types=[
            pltpu.VMEM((NCHUNK, CHUNK), jnp.int32),
            pltpu.VMEM((EPW, C), jnp.float32),
            pltpu.VMEM_SHARED((N, C), jnp.float32),
        ],
    )
    def k(rows_a, rows_b, idx_hbm, zeros_hbm, out_hbm, idx_v, rows_v, acc_sh):
        cid = lax.axis_index("c")
        sid = lax.axis_index("s")
        wid = sid * NC + cid
        pltpu.sync_copy(idx_hbm.at[wid], idx_v)
        for half, rows_hbm in enumerate((rows_a, rows_b)):
            pltpu.sync_copy(zeros_hbm.at[pl.ds(sid * NPT, NPT)],
                            acc_sh.at[pl.ds(sid * NPT, NPT)])
            pltpu.sync_copy(rows_hbm.at[pl.ds(wid * EPW, EPW)], rows_v)
            plsc.subcore_barrier()
            for j in range(NCHUNK):
                pltpu.sync_copy(rows_v.at[pl.ds(j * CHUNK, CHUNK)],
                                acc_sh.at[idx_v.at[j]], add=True)
            plsc.subcore_barrier()
            pltpu.sync_copy(acc_sh.at[pl.ds(sid * NPT, NPT)],
                            out_hbm.at[cid, half, pl.ds(sid * NPT, NPT)])

    return k


def _msg_body(xg_ref, ea_ref, wa_ref, ba_ref, w2_ref, bb_ref, ex_ref,
              *out_refs, D):
    h = jnp.maximum(ea_ref[...] @ wa_ref[...] + ba_ref[...], 0.0)
    xgf = xg_ref[...]
    xg = xgf[:, :D]
    acc = xg @ bb_ref[...] + ex_ref[...]
    R = 128 // D
    chunks = []
    for j in range(K // R):
        hh = h[:, j * R:(j + 1) * R]
        if R > 1:
            hh = jnp.repeat(hh, D, axis=1)
        chunks.append((xgf * hh).astype(jnp.bfloat16))
    y = jnp.concatenate(chunks, axis=1)
    acc = acc + lax.dot(y, w2_ref[...], preferred_element_type=jnp.float32)
    for i, o in enumerate(out_refs):
        o[...] = acc[:, i * 128:(i + 1) * 128]


def _msg(xg, ea, Wa, ba, W2, Bb, extra, Cp, D):
    EB = 1600
    nout = Cp // 128
    return pl.pallas_call(
        functools.partial(_msg_body, D=D),
        grid=(E // EB,),
        in_specs=[
            pl.BlockSpec((EB, 128), lambda i: (i, 0)),
            pl.BlockSpec((EB, 3), lambda i: (i, 0)),
            pl.BlockSpec((3, K), lambda i: (0, 0)),
            pl.BlockSpec((1, K), lambda i: (0, 0)),
            pl.BlockSpec((K * D, Cp), lambda i: (0, 0)),
            pl.BlockSpec((D, Cp), lambda i: (0, 0)),
            pl.BlockSpec((1, Cp), lambda i: (0, 0)),
        ],
        out_specs=[pl.BlockSpec((EB, 128), lambda i: (i, 0))] * nout,
        out_shape=[jax.ShapeDtypeStruct((E, 128), jnp.float32)] * nout,
    )(xg, ea, Wa, ba, W2, Bb, extra)


def _finalize_body(*refs, C, nparts):
    parts = refs[:nparts]
    c_ref, x_ref, root_ref, bias_ref, g_ref, b_ref, out_ref = refs[nparts:]
    s = jnp.concatenate([p[0] + p[1] for p in parts], axis=1)[:, :C]
    cnt = (c_ref[0] + c_ref[1])[:, CNTCOL:CNTCOL + 1]
    z = s / jnp.maximum(cnt, 1.0) + x_ref[...] @ root_ref[...] + bias_ref[...]
    mu = jnp.mean(z, axis=0, keepdims=True)
    var = jnp.mean((z - mu) ** 2, axis=0, keepdims=True)
    zn = g_ref[...] * (z - mu) * lax.rsqrt(var + 1e-5) + b_ref[...]
    out_ref[...] = jnp.maximum(zn, 0.0)


def _finalize(parts, cntp, x_nodes, root, bias, gamma, beta, C):
    return pl.pallas_call(
        functools.partial(_finalize_body, C=C, nparts=len(parts)),
        out_shape=jax.ShapeDtypeStruct((N, C), jnp.float32),
    )(*parts, cntp, x_nodes, root, bias, gamma, beta)


def _fin3_head_body(p_ref, c_ref, x_ref, root_ref, bias_ref, g_ref, b3_ref,
                    batch_ref, atom_ref, wm_ref, bm_ref, out_ref, ne_ref):
    p = p_ref[...]
    s = jnp.concatenate([p[0, 0] + p[1, 0], p[0, 1] + p[1, 1]], axis=1)
    cnt = (c_ref[0] + c_ref[1])[:, CNTCOL:CNTCOL + 1]
    z = s / jnp.maximum(cnt, 1.0) + x_ref[...] @ root_ref[...] + bias_ref[...]
    mu = jnp.mean(z, axis=0, keepdims=True)
    var = jnp.mean((z - mu) ** 2, axis=0, keepdims=True)
    h = jnp.maximum(g_ref[...] * (z - mu) * lax.rsqrt(var + 1e-5)
                    + b3_ref[...], 0.0)
    b = batch_ref[...]
    gids = lax.broadcasted_iota(jnp.int32, (G, N), 0)
    member = (b == gids)
    onehot = member.astype(jnp.float32)
    sizes_i = jnp.sum(member.astype(jnp.int32), axis=1, keepdims=True)
    gsum = onehot @ h
    graph_emb = gsum / jnp.maximum(sizes_i.astype(jnp.float32), 1.0)
    cum = jnp.sum((b < gids).astype(jnp.int32), axis=1, keepdims=True)
    mod = jnp.minimum(atom_ref[...] + cum, N - 1)
    nids = lax.broadcasted_iota(jnp.int32, (G, N), 1)
    node_emb = (nids == mod).astype(jnp.float32) @ h
    hn = 0.5 * graph_emb + node_emb
    out_ref[...] = hn @ wm_ref[...] + bm_ref[...]
    ne_ref[...] = node_emb


def _fin3_head(part3, cntp, h2, root, bias, gamma, beta, batch2, atom2,
               Wm, bm2):
    return pl.pallas_call(
        _fin3_head_body,
        out_shape=(
            jax.ShapeDtypeStruct((G, 200), jnp.float32),
            jax.ShapeDtypeStruct((G, 256), jnp.float32),
        ),
    )(part3, cntp, h2, root, bias, gamma, beta, batch2, atom2, Wm, bm2)


def _head_body(h_ref, batch_ref, atom_ref, wm_ref, bm_ref, out_ref, ne_ref):
    h = h_ref[...]
    b = batch_ref[...]
    gids = lax.broadcasted_iota(jnp.int32, (G, N), 0)
    member = (b == gids)
    onehot = member.astype(jnp.float32)
    sizes_i = jnp.sum(member.astype(jnp.int32), axis=1, keepdims=True)
    gsum = onehot @ h
    graph_emb = gsum / jnp.maximum(sizes_i.astype(jnp.float32), 1.0)
    cum = jnp.sum((b < gids).astype(jnp.int32), axis=1, keepdims=True)
    mod = jnp.minimum(atom_ref[...] + cum, N - 1)
    nids = lax.broadcasted_iota(jnp.int32, (G, N), 1)
    node_emb = (nids == mod).astype(jnp.float32) @ h
    hn = 0.5 * graph_emb + node_emb
    out_ref[...] = hn @ wm_ref[...] + bm_ref[...]
    ne_ref[...] = node_emb


def _head(h3, batch2, atom2, Wm, bm2):
    return pl.pallas_call(
        _head_body,
        out_shape=(
            jax.ShapeDtypeStruct((G, 200), jnp.float32),
            jax.ShapeDtypeStruct((G, 256), jnp.float32),
        ),
    )(h3, batch2, atom2, Wm, bm2)


def kernel(x, edge_index, edge_attr, batch, atom_num, W1a, b1a, W1b, b1b,
           root1, bias1, g1, be1, W2a, b2a, W2b, b2b, root2, bias2, g2, be2,
           W3a, b3a, W3b, b3b, root3, bias3, g3, be3, Wm, bm):
    f32 = jnp.float32
    src2 = edge_index[0].reshape(NW, NCHUNK, CHUNK)
    dst2 = edge_index[1].reshape(NW, NCHUNK, CHUNK)

    bf16 = jnp.bfloat16
    W2_1 = jnp.pad(W1b.reshape(K, 15, 64), ((0, 0), (0, 1), (0, 64)))
    W2_1 = W2_1.reshape(K * 16, 128).astype(bf16)
    Bb1 = jnp.pad(b1b.reshape(15, 64), ((0, 1), (0, 64)))
    W2_2 = W2b.reshape(K * 64, 128).astype(bf16)
    Bb2 = b2b.reshape(64, 128)
    W2_3 = W3b.reshape(K * 128, 256).astype(bf16)
    Bb3 = b3b.reshape(128, 256)

    cnt_row = jnp.zeros((1, 128), f32).at[0, CNTCOL].set(1.0)
    z128 = jnp.zeros((1, 128), f32)
    z256 = jnp.zeros((1, 256), f32)
    zN128 = jnp.zeros((N, 128), f32)

    xp1 = jnp.tile(jnp.pad(x, ((0, 0), (0, 1))), (1, 8))
    scat = _sc_scatter_add(128)

    xg1 = _sc_gather(128)(xp1, src2)
    (m1,) = _msg(xg1, edge_attr, W1a, b1a.reshape(1, K), W2_1, Bb1,
                 cnt_row, 128, 16)
    part1 = scat(m1, dst2, zN128)
    h1 = _finalize([part1], part1, x, root1, bias1.reshape(1, 64),
                   g1.reshape(1, 64), be1.reshape(1, 64), 64)

    h1p = jnp.concatenate([h1, h1], axis=1)
    xg2 = _sc_gather(128)(h1p, src2)
    (m2,) = _msg(xg2, edge_attr, W2a, b2a.reshape(1, K), W2_2, Bb2,
                 z128, 128, 64)
    part2 = scat(m2, dst2, zN128)
    h2 = _finalize([part2], part1, h1, root2, bias2.reshape(1, 128),
                   g2.reshape(1, 128), be2.reshape(1, 128), 128)

    xg3 = _sc_gather(128)(h2, src2)
    m3a, m3b = _msg(xg3, edge_attr, W3a, b3a.reshape(1, K), W2_3, Bb3,
                    z256, 256, 128)
    part3 = _sc_scatter_add2(128)(m3a, m3b, dst2, zN128)
    out, node_emb = _fin3_head(part3, part1, h2, root3, bias3.reshape(1, 256),
                               g3.reshape(1, 256), be3.reshape(1, 256),
                               batch.reshape(1, N), atom_num.reshape(G, 1),
                               Wm, bm.reshape(1, 200))
    return (out, node_emb)


# speedup vs baseline: 1.7155x; 1.7155x over previous
import functools

import jax
import jax.numpy as jnp
from jax import lax
from jax.experimental import pallas as pl
from jax.experimental.pallas import tpu as pltpu
from jax.experimental.pallas import tpu_sc as plsc

N = 3200
E = 12800
G = 128
K = 32

NC = 2
NS = 16
NW = NC * NS
E2 = E // 2
EPW = E2 // NW
CHUNK = 40
NCHUNK = EPW // CHUNK
NPT = N // NS
CNTCOL = 64
EB = 1600

_MESH = plsc.VectorSubcoreMesh(core_axis_name="c", subcore_axis_name="s")


def _sc_gather(D):
    @functools.partial(
        pl.kernel,
        out_type=jax.ShapeDtypeStruct((E2, D), jnp.float32),
        mesh=_MESH,
        scratch_types=[
            pltpu.VMEM((NCHUNK, CHUNK), jnp.int32),
            pltpu.VMEM((EPW, D), jnp.float32),
            pltpu.SemaphoreType.DMA,
        ],
    )
    def k(table_hbm, idx_hbm, out_hbm, idx_v, rows_v, sem):
        wid = lax.axis_index("s") * NC + lax.axis_index("c")
        pltpu.sync_copy(idx_hbm.at[wid], idx_v)
        copies = [
            pltpu.async_copy(
                table_hbm.at[idx_v.at[j]],
                rows_v.at[pl.ds(j * CHUNK, CHUNK)],
                sem,
            )
            for j in range(NCHUNK)
        ]
        for c in copies:
            c.wait()
        pltpu.sync_copy(rows_v, out_hbm.at[pl.ds(wid * EPW, EPW)])

    return k


def _sc_scatter_add(C):
    @functools.partial(
        pl.kernel,
        out_type=pltpu.HBM((NC, N, C), jnp.float32),
        mesh=_MESH,
        scratch_types=[
            pltpu.VMEM((NCHUNK, CHUNK), jnp.int32),
            pltpu.VMEM((EPW, C), jnp.float32),
            pltpu.VMEM_SHARED((N, C), jnp.float32),
        ],
    )
    def k(rows_hbm, idx_hbm, zeros_hbm, out_hbm, idx_v, rows_v, acc_sh):
        cid = lax.axis_index("c")
        sid = lax.axis_index("s")
        wid = sid * NC + cid
        pltpu.sync_copy(zeros_hbm.at[pl.ds(sid * NPT, NPT)],
                        acc_sh.at[pl.ds(sid * NPT, NPT)])
        pltpu.sync_copy(idx_hbm.at[wid], idx_v)
        pltpu.sync_copy(rows_hbm.at[pl.ds(wid * EPW, EPW)], rows_v)
        plsc.subcore_barrier()
        for j in range(NCHUNK):
            pltpu.sync_copy(rows_v.at[pl.ds(j * CHUNK, CHUNK)],
                            acc_sh.at[idx_v.at[j]], add=True)
        plsc.subcore_barrier()
        pltpu.sync_copy(acc_sh.at[pl.ds(sid * NPT, NPT)],
                        out_hbm.at[cid, pl.ds(sid * NPT, NPT)])

    return k


def _sc_scatter_add2(C):
    @functools.partial(
        pl.kernel,
        out_type=pltpu.HBM((NC, 2, N, C), jnp.float32),
        mesh=_MESH,
        scratch_types=[
            pltpu.VMEM((NCHUNK, CHUNK), jnp.int32),
            pltpu.VMEM((EPW, C), jnp.float32),
            pltpu.VMEM_SHARED((N, C), jnp.float32),
        ],
    )
    def k(rows_a, rows_b, idx_hbm, zeros_hbm, out_hbm, idx_v, rows_v, acc_sh):
        cid = lax.axis_index("c")
        sid = lax.axis_index("s")
        wid = sid * NC + cid
        pltpu.sync_copy(idx_hbm.at[wid], idx_v)
        for half, rows_hbm in enumerate((rows_a, rows_b)):
            pltpu.sync_copy(zeros_hbm.at[pl.ds(sid * NPT, NPT)],
                            acc_sh.at[pl.ds(sid * NPT, NPT)])
            pltpu.sync_copy(rows_hbm.at[pl.ds(wid * EPW, EPW)], rows_v)
            plsc.subcore_barrier()
            for j in range(NCHUNK):
                pltpu.sync_copy(rows_v.at[pl.ds(j * CHUNK, CHUNK)],
                                acc_sh.at[idx_v.at[j]], add=True)
            plsc.subcore_barrier()
            pltpu.sync_copy(acc_sh.at[pl.ds(sid * NPT, NPT)],
                            out_hbm.at[cid, half, pl.ds(sid * NPT, NPT)])

    return k


def _msg_body(xg_ref, ea_ref, wa_ref, ba_ref, w2_ref, bb_ref, ex_ref,
              *out_refs, D):
    h = jnp.maximum(ea_ref[...] @ wa_ref[...] + ba_ref[...], 0.0)
    xg = xg_ref[...][:, :D]
    acc = xg @ bb_ref[...] + ex_ref[...]
    if D == 128:
        y = jnp.concatenate(
            [(xg * h[:, k:k + 1]).astype(jnp.bfloat16) for k in range(K)],
            axis=1)
        acc = acc + lax.dot(y, w2_ref[...],
                            preferred_element_type=jnp.float32)
    else:
        for k in range(K):
            acc = acc + (xg * h[:, k:k + 1]) @ w2_ref[pl.ds(k * D, D), :]
    for i, o in enumerate(out_refs):
        o[...] = acc[:, i * 128:(i + 1) * 128]


def _msg(xg, ea, Wa, ba, W2, Bb, extra, Cp, D):
    nout = Cp // 128
    return pl.pallas_call(
        functools.partial(_msg_body, D=D),
        grid=(E2 // EB,),
        in_specs=[
            pl.BlockSpec((EB, 128), lambda i: (i, 0)),
            pl.BlockSpec((EB, 3), lambda i: (i, 0)),
            pl.BlockSpec((3, K), lambda i: (0, 0)),
            pl.BlockSpec((1, K), lambda i: (0, 0)),
            pl.BlockSpec((K * D, Cp), lambda i: (0, 0)),
            pl.BlockSpec((D, Cp), lambda i: (0, 0)),
            pl.BlockSpec((1, Cp), lambda i: (0, 0)),
        ],
        out_specs=[pl.BlockSpec((EB, 128), lambda i: (i, 0))] * nout,
        out_shape=[jax.ShapeDtypeStruct((E2, 128), jnp.float32)] * nout,
    )(xg, ea, Wa, ba, W2, Bb, extra)


def _finalize_body(pa_ref, pb_ref, ca_ref, cb_ref, x_ref, root_ref, bias_ref,
                   g_ref, b_ref, out_ref):
    s = pa_ref[0] + pa_ref[1] + pb_ref[0] + pb_ref[1]
    cnt = (ca_ref[0] + ca_ref[1] + cb_ref[0]
           + cb_ref[1])[:, CNTCOL:CNTCOL + 1]
    C = root_ref.shape[1]
    z = (s[:, :C] / jnp.maximum(cnt, 1.0)
         + x_ref[...] @ root_ref[...] + bias_ref[...])
    mu = jnp.mean(z, axis=0, keepdims=True)
    var = jnp.mean((z - mu) ** 2, axis=0, keepdims=True)
    zn = g_ref[...] * (z - mu) * lax.rsqrt(var + 1e-5) + b_ref[...]
    out_ref[...] = jnp.maximum(zn, 0.0)


def _finalize(pa, pb, ca, cb, x_nodes, root, bias, gamma, beta, C):
    return pl.pallas_call(
        _finalize_body,
        out_shape=jax.ShapeDtypeStruct((N, C), jnp.float32),
    )(pa, pb, ca, cb, x_nodes, root, bias, gamma, beta)


def _fin3_head_body(pa_ref, pb_ref, ca_ref, cb_ref, x_ref, root_ref, bias_ref,
                    g_ref, b3_ref, batch_ref, atom_ref, wm_ref, bm_ref,
                    out_ref, ne_ref):
    pa = pa_ref[...]
    pb = pb_ref[...]
    s = jnp.concatenate(
        [pa[0, 0] + pa[1, 0] + pb[0, 0] + pb[1, 0],
         pa[0, 1] + pa[1, 1] + pb[0, 1] + pb[1, 1]], axis=1)
    cnt = (ca_ref[0] + ca_ref[1] + cb_ref[0]
           + cb_ref[1])[:, CNTCOL:CNTCOL + 1]
    z = (s / jnp.maximum(cnt, 1.0)
         + x_ref[...] @ root_ref[...] + bias_ref[...])
    mu = jnp.mean(z, axis=0, keepdims=True)
    var = jnp.mean((z - mu) ** 2, axis=0, keepdims=True)
    h = jnp.maximum(g_ref[...] * (z - mu) * lax.rsqrt(var + 1e-5)
                    + b3_ref[...], 0.0)
    b = batch_ref[...]
    gids = lax.broadcasted_iota(jnp.int32, (G, N), 0)
    member = (b == gids)
    onehot = member.astype(jnp.float32)
    sizes_i = jnp.sum(member.astype(jnp.int32), axis=1, keepdims=True)
    gsum = onehot @ h
    graph_emb = gsum / jnp.maximum(sizes_i.astype(jnp.float32), 1.0)
    cum = jnp.sum((b < gids).astype(jnp.int32), axis=1, keepdims=True)
    mod = jnp.minimum(atom_ref[...] + cum, N - 1)
    nids = lax.broadcasted_iota(jnp.int32, (G, N), 1)
    node_emb = (nids == mod).astype(jnp.float32) @ h
    hn = 0.5 * graph_emb + node_emb
    out_ref[...] = hn @ wm_ref[...] + bm_ref[...]
    ne_ref[...] = node_emb


def _fin3_head(pa, pb, ca, cb, h2, root, bias, gamma, beta, batch2, atom2,
               Wm, bm2):
    return pl.pallas_call(
        _fin3_head_body,
        out_shape=(
            jax.ShapeDtypeStruct((G, 200), jnp.float32),
            jax.ShapeDtypeStruct((G, 256), jnp.float32),
        ),
    )(pa, pb, ca, cb, h2, root, bias, gamma, beta, batch2, atom2, Wm, bm2)


def kernel(x, edge_index, edge_attr, batch, atom_num, W1a, b1a, W1b, b1b,
           root1, bias1, g1, be1, W2a, b2a, W2b, b2b, root2, bias2, g2, be2,
           W3a, b3a, W3b, b3b, root3, bias3, g3, be3, Wm, bm):
    f32 = jnp.float32
    src2 = edge_index[0].reshape(2, NW, NCHUNK, CHUNK)
    dst2 = edge_index[1].reshape(2, NW, NCHUNK, CHUNK)
    eah = edge_attr.reshape(2, E2, 3)

    bf16 = jnp.bfloat16
    W2_1 = jnp.pad(W1b.reshape(K, 15, 64), ((0, 0), (0, 1), (0, 64)))
    W2_1 = W2_1.reshape(K * 16, 128)
    Bb1 = jnp.pad(b1b.reshape(15, 64), ((0, 1), (0, 64)))
    W2_2 = W2b.reshape(K * 64, 128)
    Bb2 = b2b.reshape(64, 128)
    W2_3 = W3b.reshape(K * 128, 256).astype(bf16)
    Bb3 = b3b.reshape(128, 256)

    cnt_row = jnp.zeros((1, 128), f32).at[0, CNTCOL].set(1.0)
    z128 = jnp.zeros((1, 128), f32)
    z256 = jnp.zeros((1, 256), f32)
    zN128 = jnp.zeros((N, 128), f32)

    xp1 = jnp.pad(x, ((0, 0), (0, 113)))
    gat = _sc_gather(128)
    scat = _sc_scatter_add(128)
    scat2 = _sc_scatter_add2(128)

    def layer(table, Wa, ba, W2, Bb, extra, Cp, D):
        xgA = gat(table, src2[0])
        xgB = gat(table, src2[1])
        msA = _msg(xgA, eah[0], Wa, ba.reshape(1, K), W2, Bb, extra, Cp, D)
        msB = _msg(xgB, eah[1], Wa, ba.reshape(1, K), W2, Bb, extra, Cp, D)
        if Cp == 128:
            pA = scat(msA[0], dst2[0], zN128)
            pB = scat(msB[0], dst2[1], zN128)
        else:
            pA = scat2(msA[0], msA[1], dst2[0], zN128)
            pB = scat2(msB[0], msB[1], dst2[1], zN128)
        return pA, pB

    c1A, c1B = layer(xp1, W1a, b1a, W2_1, Bb1, cnt_row, 128, 16)
    h1 = _finalize(c1A, c1B, c1A, c1B, x, root1, bias1.reshape(1, 64),
                   g1.reshape(1, 64), be1.reshape(1, 64), 64)

    h1p = jnp.pad(h1, ((0, 0), (0, 64)))
    p2A, p2B = layer(h1p, W2a, b2a, W2_2, Bb2, z128, 128, 64)
    h2 = _finalize(p2A, p2B, c1A, c1B, h1, root2, bias2.reshape(1, 128),
                   g2.reshape(1, 128), be2.reshape(1, 128), 128)

    p3A, p3B = layer(h2, W3a, b3a, W2_3, Bb3, z256, 256, 128)
    out, node_emb = _fin3_head(p3A, p3B, c1A, c1B, h2, root3,
                               bias3.reshape(1, 256), g3.reshape(1, 256),
                               be3.reshape(1, 256), batch.reshape(1, N),
                               atom_num.reshape(G, 1), Wm, bm.reshape(1, 200))
    return (out, node_emb)


# speedup vs baseline: 1.8177x; 1.0596x over previous
import functools

import jax
import jax.numpy as jnp
from jax import lax
from jax.experimental import pallas as pl
from jax.experimental.pallas import tpu as pltpu
from jax.experimental.pallas import tpu_sc as plsc

N = 3200
E = 12800
G = 128
K = 32

NC = 2
NS = 16
NW = NC * NS
EPW = E // NW
CHUNK = 80
NCHUNK = EPW // CHUNK
NPT = N // NS
CNTCOL = 64

_MESH = plsc.VectorSubcoreMesh(core_axis_name="c", subcore_axis_name="s")


def _sc_gather(D):
    @functools.partial(
        pl.kernel,
        out_type=jax.ShapeDtypeStruct((E, D), jnp.float32),
        mesh=_MESH,
        scratch_types=[
            pltpu.VMEM((NCHUNK, CHUNK), jnp.int32),
            pltpu.VMEM((EPW, D), jnp.float32),
            pltpu.SemaphoreType.DMA,
        ],
    )
    def k(table_hbm, idx_hbm, out_hbm, idx_v, rows_v, sem):
        wid = lax.axis_index("s") * NC + lax.axis_index("c")
        pltpu.sync_copy(idx_hbm.at[wid], idx_v)
        copies = [
            pltpu.async_copy(
                table_hbm.at[idx_v.at[j]],
                rows_v.at[pl.ds(j * CHUNK, CHUNK)],
                sem,
            )
            for j in range(NCHUNK)
        ]
        for c in copies:
            c.wait()
        pltpu.sync_copy(rows_v, out_hbm.at[pl.ds(wid * EPW, EPW)])

    return k


def _sc_scatter_add(C):
    @functools.partial(
        pl.kernel,
        out_type=pltpu.HBM((NC, N, C), jnp.float32),
        mesh=_MESH,
        scratch_types=[
            pltpu.VMEM((NCHUNK, CHUNK), jnp.int32),
            pltpu.VMEM((EPW, C), jnp.float32),
            pltpu.VMEM_SHARED((N, C), jnp.float32),
        ],
    )
    def k(rows_hbm, idx_hbm, zeros_hbm, out_hbm, idx_v, rows_v, acc_sh):
        cid = lax.axis_index("c")
        sid = lax.axis_index("s")
        wid = sid * NC + cid
        pltpu.sync_copy(zeros_hbm.at[pl.ds(sid * NPT, NPT)],
                        acc_sh.at[pl.ds(sid * NPT, NPT)])
        pltpu.sync_copy(idx_hbm.at[wid], idx_v)
        pltpu.sync_copy(rows_hbm.at[pl.ds(wid * EPW, EPW)], rows_v)
        plsc.subcore_barrier()
        for j in range(NCHUNK):
            pltpu.sync_copy(rows_v.at[pl.ds(j * CHUNK, CHUNK)],
                            acc_sh.at[idx_v.at[j]], add=True)
        plsc.subcore_barrier()
        pltpu.sync_copy(acc_sh.at[pl.ds(sid * NPT, NPT)],
                        out_hbm.at[cid, pl.ds(sid * NPT, NPT)])

    return k


def _sc_scatter_add2(C):
    @functools.partial(
        pl.kernel,
        out_type=pltpu.HBM((NC, 2, N, C), jnp.float32),
        mesh=_MESH,
        scratch_types=[
            pltpu.VMEM((NCHUNK, CHUNK), jnp.int32),
            pltpu.VMEM((EPW, C), jnp.float32),
            pltpu.VMEM_SHARED((N, C), jnp.float32),
        ],
    )
    def k(rows_a, rows_b, idx_hbm, zeros_hbm, out_hbm, idx_v, rows_v, acc_sh):
        cid = lax.axis_index("c")
        sid = lax.axis_index("s")
        wid = sid * NC + cid
        pltpu.sync_copy(idx_hbm.at[wid], idx_v)
        for half, rows_hbm in enumerate((rows_a, rows_b)):
            pltpu.sync_copy(zeros_hbm.at[pl.ds(sid * NPT, NPT)],
                            acc_sh.at[pl.ds(sid * NPT, NPT)])
            pltpu.sync_copy(rows_hbm.at[pl.ds(wid * EPW, EPW)], rows_v)
            plsc.subcore_barrier()
            for j in range(NCHUNK):
                pltpu.sync_copy(rows_v.at[pl.ds(j * CHUNK, CHUNK)],
                                acc_sh.at[idx_v.at[j]], add=True)
            plsc.subcore_barrier()
            pltpu.sync_copy(acc_sh.at[pl.ds(sid * NPT, NPT)],
                            out_hbm.at[cid, half, pl.ds(sid * NPT, NPT)])

    return k


def _msg_body(xg_ref, ea_ref, wa_ref, ba_ref, w2_ref, bb_ref, ex_ref,
              *out_refs, D):
    h = jnp.maximum(ea_ref[...] @ wa_ref[...] + ba_ref[...], 0.0)
    xg = xg_ref[...][:, :D]
    acc = xg @ bb_ref[...] + ex_ref[...]
    if D == 128:
        y = jnp.concatenate(
            [(xg * h[:, k:k + 1]).astype(jnp.bfloat16) for k in range(K)],
            axis=1)
        acc = acc + lax.dot(y, w2_ref[...],
                            preferred_element_type=jnp.float32)
    else:
        for k in range(K):
            acc = acc + (xg * h[:, k:k + 1]) @ w2_ref[pl.ds(k * D, D), :]
    for i, o in enumerate(out_refs):
        o[...] = acc[:, i * 128:(i + 1) * 128]


def _msg(xg, ea, Wa, ba, W2, Bb, extra, Cp, D):
    EB = 1600
    nout = Cp // 128
    return pl.pallas_call(
        functools.partial(_msg_body, D=D),
        grid=(E // EB,),
        in_specs=[
            pl.BlockSpec((EB, 128), lambda i: (i, 0)),
            pl.BlockSpec((EB, 3), lambda i: (i, 0)),
            pl.BlockSpec((3, K), lambda i: (0, 0)),
            pl.BlockSpec((1, K), lambda i: (0, 0)),
            pl.BlockSpec((K * D, Cp), lambda i: (0, 0)),
            pl.BlockSpec((D, Cp), lambda i: (0, 0)),
            pl.BlockSpec((1, Cp), lambda i: (0, 0)),
        ],
        out_specs=[pl.BlockSpec((EB, 128), lambda i: (i, 0))] * nout,
        out_shape=[jax.ShapeDtypeStruct((E, 128), jnp.float32)] * nout,
    )(xg, ea, Wa, ba, W2, Bb, extra)


def _finalize_body(p_ref, c_ref, x_ref, root_ref, bias_ref, g_ref, b_ref,
                   out_ref):
    C = root_ref.shape[1]
    s = (p_ref[0] + p_ref[1])[:, :C]
    cnt = (c_ref[0] + c_ref[1])[:, CNTCOL:CNTCOL + 1]
    z = s / jnp.maximum(cnt, 1.0) + x_ref[...] @ root_ref[...] + bias_ref[...]
    mu = jnp.mean(z, axis=0, keepdims=True)
    var = jnp.mean((z - mu) ** 2, axis=0, keepdims=True)
    zn = g_ref[...] * (z - mu) * lax.rsqrt(var + 1e-5) + b_ref[...]
    out_ref[...] = jnp.maximum(zn, 0.0)


def _finalize(part, cntp, x_nodes, root, bias, gamma, beta, C):
    return pl.pallas_call(
        _finalize_body,
        out_shape=jax.ShapeDtypeStruct((N, C), jnp.float32),
    )(part, cntp, x_nodes, root, bias, gamma, beta)


def _fin3_head_body(p_ref, c_ref, x_ref, root_ref, bias_ref, g_ref, b3_ref,
                    batch_ref, atom_ref, wm_ref, bm_ref, out_ref, ne_ref):
    p = p_ref[...]
    s = jnp.concatenate([p[0, 0] + p[1, 0], p[0, 1] + p[1, 1]], axis=1)
    cnt = (c_ref[0] + c_ref[1])[:, CNTCOL:CNTCOL + 1]
    z = s / jnp.maximum(cnt, 1.0) + x_ref[...] @ root_ref[...] + bias_ref[...]
    mu = jnp.mean(z, axis=0, keepdims=True)
    var = jnp.mean((z - mu) ** 2, axis=0, keepdims=True)
    h = jnp.maximum(g_ref[...] * (z - mu) * lax.rsqrt(var + 1e-5)
                    + b3_ref[...], 0.0)
    b = batch_ref[...]
    gids = lax.broadcasted_iota(jnp.int32, (G, N), 0)
    member = (b == gids)
    onehot = member.astype(jnp.float32)
    sizes_i = jnp.sum(member.astype(jnp.int32), axis=1, keepdims=True)
    gsum = onehot @ h
    graph_emb = gsum / jnp.maximum(sizes_i.astype(jnp.float32), 1.0)
    cum = jnp.sum((b < gids).astype(jnp.int32), axis=1, keepdims=True)
    mod = jnp.minimum(atom_ref[...] + cum, N - 1)
    nids = lax.broadcasted_iota(jnp.int32, (G, N), 1)
    node_emb = (nids == mod).astype(jnp.float32) @ h
    hn = 0.5 * graph_emb + node_emb
    out_ref[...] = hn @ wm_ref[...] + bm_ref[...]
    ne_ref[...] = node_emb


def _fin3_head(part3, cntp, h2, root, bias, gamma, beta, batch2, atom2,
               Wm, bm2):
    return pl.pallas_call(
        _fin3_head_body,
        out_shape=(
            jax.ShapeDtypeStruct((G, 200), jnp.float32),
            jax.ShapeDtypeStruct((G, 256), jnp.float32),
        ),
    )(part3, cntp, h2, root, bias, gamma, beta, batch2, atom2, Wm, bm2)


def kernel(x, edge_index, edge_attr, batch, atom_num, W1a, b1a, W1b, b1b,
           root1, bias1, g1, be1, W2a, b2a, W2b, b2b, root2, bias2, g2, be2,
           W3a, b3a, W3b, b3b, root3, bias3, g3, be3, Wm, bm):
    f32 = jnp.float32
    src2 = edge_index[0].reshape(NW, NCHUNK, CHUNK)
    dst2 = edge_index[1].reshape(NW, NCHUNK, CHUNK)

    bf16 = jnp.bfloat16
    W2_1 = jnp.pad(W1b.reshape(K, 15, 64), ((0, 0), (0, 1), (0, 64)))
    W2_1 = W2_1.reshape(K * 16, 128)
    Bb1 = jnp.pad(b1b.reshape(15, 64), ((0, 1), (0, 64)))
    W2_2 = W2b.reshape(K * 64, 128)
    Bb2 = b2b.reshape(64, 128)
    W2_3 = W3b.reshape(K * 128, 256).astype(bf16)
    Bb3 = b3b.reshape(128, 256)

    cnt_row = jnp.zeros((1, 128), f32).at[0, CNTCOL].set(1.0)
    z128 = jnp.zeros((1, 128), f32)
    z256 = jnp.zeros((1, 256), f32)
    zN128 = jnp.zeros((N, 128), f32)

    xp1 = jnp.pad(x, ((0, 0), (0, 113)))
    scat = _sc_scatter_add(128)

    xg1 = _sc_gather(128)(xp1, src2)
    (m1,) = _msg(xg1, edge_attr, W1a, b1a.reshape(1, K), W2_1, Bb1,
                 cnt_row, 128, 16)
    part1 = scat(m1, dst2, zN128)
    h1 = _finalize(part1, part1, x, root1, bias1.reshape(1, 64),
                   g1.reshape(1, 64), be1.reshape(1, 64), 64)

    h1p = jnp.pad(h1, ((0, 0), (0, 64)))
    xg2 = _sc_gather(128)(h1p, src2)
    (m2,) = _msg(xg2, edge_attr, W2a, b2a.reshape(1, K), W2_2, Bb2,
                 z128, 128, 64)
    part2 = scat(m2, dst2, zN128)
    h2 = _finalize(part2, part1, h1, root2, bias2.reshape(1, 128),
                   g2.reshape(1, 128), be2.reshape(1, 128), 128)

    xg3 = _sc_gather(128)(h2, src2)
    m3a, m3b = _msg(xg3, edge_attr, W3a, b3a.reshape(1, K), W2_3, Bb3,
                    z256, 256, 128)
    part3 = _sc_scatter_add2(128)(m3a, m3b, dst2, zN128)
    out, node_emb = _fin3_head(part3, part1, h2, root3, bias3.reshape(1, 256),
                               g3.reshape(1, 256), be3.reshape(1, 256),
                               batch.reshape(1, N), atom_num.reshape(G, 1),
                               Wm, bm.reshape(1, 200))
    return (out, node_emb)


# speedup vs baseline: 1.9119x; 1.0518x over previous
import functools

import jax
import jax.numpy as jnp
from jax import lax
from jax.experimental import pallas as pl
from jax.experimental.pallas import tpu as pltpu
from jax.experimental.pallas import tpu_sc as plsc

N = 3200
E = 12800
G = 128
K = 32

NC = 2
NS = 16
NW = NC * NS
EPW = E // NW
CHUNK = 80
NCHUNK = EPW // CHUNK
NPT = N // NS
CNTCOL = 64

_MESH = plsc.VectorSubcoreMesh(core_axis_name="c", subcore_axis_name="s")


def _sc_gather(D, dtype=jnp.float32):
    @functools.partial(
        pl.kernel,
        out_type=jax.ShapeDtypeStruct((E, D), dtype),
        mesh=_MESH,
        scratch_types=[
            pltpu.VMEM((NCHUNK, CHUNK), jnp.int32),
            pltpu.VMEM((EPW, D), dtype),
            pltpu.SemaphoreType.DMA,
        ],
    )
    def k(table_hbm, idx_hbm, out_hbm, idx_v, rows_v, sem):
        wid = lax.axis_index("s") * NC + lax.axis_index("c")
        pltpu.sync_copy(idx_hbm.at[wid], idx_v)
        copies = [
            pltpu.async_copy(
                table_hbm.at[idx_v.at[j]],
                rows_v.at[pl.ds(j * CHUNK, CHUNK)],
                sem,
            )
            for j in range(NCHUNK)
        ]
        for c in copies:
            c.wait()
        pltpu.sync_copy(rows_v, out_hbm.at[pl.ds(wid * EPW, EPW)])

    return k


def _sc_scatter_add(C):
    @functools.partial(
        pl.kernel,
        out_type=pltpu.HBM((NC, N, C), jnp.float32),
        mesh=_MESH,
        scratch_types=[
            pltpu.VMEM((NCHUNK, CHUNK), jnp.int32),
            pltpu.VMEM((EPW, C), jnp.float32),
            pltpu.VMEM_SHARED((N, C), jnp.float32),
        ],
    )
    def k(rows_hbm, idx_hbm, zeros_hbm, out_hbm, idx_v, rows_v, acc_sh):
        cid = lax.axis_index("c")
        sid = lax.axis_index("s")
        wid = sid * NC + cid
        pltpu.sync_copy(zeros_hbm.at[pl.ds(sid * NPT, NPT)],
                        acc_sh.at[pl.ds(sid * NPT, NPT)])
        pltpu.sync_copy(idx_hbm.at[wid], idx_v)
        pltpu.sync_copy(rows_hbm.at[pl.ds(wid * EPW, EPW)], rows_v)
        plsc.subcore_barrier()
        for j in range(NCHUNK):
            pltpu.sync_copy(rows_v.at[pl.ds(j * CHUNK, CHUNK)],
                            acc_sh.at[idx_v.at[j]], add=True)
        plsc.subcore_barrier()
        pltpu.sync_copy(acc_sh.at[pl.ds(sid * NPT, NPT)],
                        out_hbm.at[cid, pl.ds(sid * NPT, NPT)])

    return k


def _sc_scatter_add2(C):
    @functools.partial(
        pl.kernel,
        out_type=pltpu.HBM((NC, 2, N, C), jnp.float32),
        mesh=_MESH,
        scratch_types=[
            pltpu.VMEM((NCHUNK, CHUNK), jnp.int32),
            pltpu.VMEM((EPW, C), jnp.float32),
            pltpu.VMEM_SHARED((N, C), jnp.float32),
        ],
    )
    def k(rows_a, rows_b, idx_hbm, zeros_hbm, out_hbm, idx_v, rows_v, acc_sh):
        cid = lax.axis_index("c")
        sid = lax.axis_index("s")
        wid = sid * NC + cid
        pltpu.sync_copy(idx_hbm.at[wid], idx_v)
        for half, rows_hbm in enumerate((rows_a, rows_b)):
            pltpu.sync_copy(zeros_hbm.at[pl.ds(sid * NPT, NPT)],
                            acc_sh.at[pl.ds(sid * NPT, NPT)])
            pltpu.sync_copy(rows_hbm.at[pl.ds(wid * EPW, EPW)], rows_v)
            plsc.subcore_barrier()
            for j in range(NCHUNK):
                pltpu.sync_copy(rows_v.at[pl.ds(j * CHUNK, CHUNK)],
                                acc_sh.at[idx_v.at[j]], add=True)
            plsc.subcore_barrier()
            pltpu.sync_copy(acc_sh.at[pl.ds(sid * NPT, NPT)],
                            out_hbm.at[cid, half, pl.ds(sid * NPT, NPT)])

    return k


def _msg_body(xg_ref, ea_ref, wa_ref, ba_ref, w2_ref, bb_ref, ex_ref,
              *out_refs, D):
    h = jnp.maximum(ea_ref[...] @ wa_ref[...] + ba_ref[...], 0.0)
    xg = xg_ref[...][:, :D].astype(jnp.float32)
    acc = xg @ bb_ref[...] + ex_ref[...]
    if D == 128:
        xgb = xg.astype(jnp.bfloat16)
        hb = h.astype(jnp.bfloat16)
        y = jnp.concatenate(
            [xgb * hb[:, k:k + 1] for k in range(K)], axis=1)
        acc = acc + lax.dot(y, w2_ref[...],
                            preferred_element_type=jnp.float32)
    else:
        for k in range(K):
            acc = acc + (xg * h[:, k:k + 1]) @ w2_ref[pl.ds(k * D, D), :]
    for i, o in enumerate(out_refs):
        o[...] = acc[:, i * 128:(i + 1) * 128]


def _msg(xg, ea, Wa, ba, W2, Bb, extra, Cp, D):
    EB = 1600
    nout = Cp // 128
    return pl.pallas_call(
        functools.partial(_msg_body, D=D),
        grid=(E // EB,),
        in_specs=[
            pl.BlockSpec((EB, 128), lambda i: (i, 0)),
            pl.BlockSpec((EB, 3), lambda i: (i, 0)),
            pl.BlockSpec((3, K), lambda i: (0, 0)),
            pl.BlockSpec((1, K), lambda i: (0, 0)),
            pl.BlockSpec((K * D, Cp), lambda i: (0, 0)),
            pl.BlockSpec((D, Cp), lambda i: (0, 0)),
            pl.BlockSpec((1, Cp), lambda i: (0, 0)),
        ],
        out_specs=[pl.BlockSpec((EB, 128), lambda i: (i, 0))] * nout,
        out_shape=[jax.ShapeDtypeStruct((E, 128), jnp.float32)] * nout,
    )(xg, ea, Wa, ba, W2, Bb, extra)


def _finalize_body(p_ref, c_ref, x_ref, root_ref, bias_ref, g_ref, b_ref,
                   out_ref):
    C = root_ref.shape[1]
    s = (p_ref[0] + p_ref[1])[:, :C]
    cnt = (c_ref[0] + c_ref[1])[:, CNTCOL:CNTCOL + 1]
    z = s / jnp.maximum(cnt, 1.0) + x_ref[...] @ root_ref[...] + bias_ref[...]
    mu = jnp.mean(z, axis=0, keepdims=True)
    var = jnp.mean((z - mu) ** 2, axis=0, keepdims=True)
    zn = g_ref[...] * (z - mu) * lax.rsqrt(var + 1e-5) + b_ref[...]
    out_ref[...] = jnp.maximum(zn, 0.0)


def _finalize(part, cntp, x_nodes, root, bias, gamma, beta, C):
    return pl.pallas_call(
        _finalize_body,
        out_shape=jax.ShapeDtypeStruct((N, C), jnp.float32),
    )(part, cntp, x_nodes, root, bias, gamma, beta)


def _fin3_head_body(p_ref, c_ref, x_ref, root_ref, bias_ref, g_ref, b3_ref,
                    batch_ref, atom_ref, wm_ref, bm_ref, out_ref, ne_ref):
    p = p_ref[...]
    s = jnp.concatenate([p[0, 0] + p[1, 0], p[0, 1] + p[1, 1]], axis=1)
    cnt = (c_ref[0] + c_ref[1])[:, CNTCOL:CNTCOL + 1]
    z = s / jnp.maximum(cnt, 1.0) + x_ref[...] @ root_ref[...] + bias_ref[...]
    mu = jnp.mean(z, axis=0, keepdims=True)
    var = jnp.mean((z - mu) ** 2, axis=0, keepdims=True)
    h = jnp.maximum(g_ref[...] * (z - mu) * lax.rsqrt(var + 1e-5)
                    + b3_ref[...], 0.0)
    b = batch_ref[...]
    gids = lax.broadcasted_iota(jnp.int32, (G, N), 0)
    member = (b == gids)
    onehot = member.astype(jnp.float32)
    sizes_i = jnp.sum(member.astype(jnp.int32), axis=1, keepdims=True)
    gsum = onehot @ h
    graph_emb = gsum / jnp.maximum(sizes_i.astype(jnp.float32), 1.0)
    cum = jnp.sum((b < gids).astype(jnp.int32), axis=1, keepdims=True)
    mod = jnp.minimum(atom_ref[...] + cum, N - 1)
    nids = lax.broadcasted_iota(jnp.int32, (G, N), 1)
    node_emb = (nids == mod).astype(jnp.float32) @ h
    hn = 0.5 * graph_emb + node_emb
    out_ref[...] = hn @ wm_ref[...] + bm_ref[...]
    ne_ref[...] = node_emb


def _fin3_head(part3, cntp, h2, root, bias, gamma, beta, batch2, atom2,
               Wm, bm2):
    return pl.pallas_call(
        _fin3_head_body,
        out_shape=(
            jax.ShapeDtypeStruct((G, 200), jnp.float32),
            jax.ShapeDtypeStruct((G, 256), jnp.float32),
        ),
    )(part3, cntp, h2, root, bias, gamma, beta, batch2, atom2, Wm, bm2)


def kernel(x, edge_index, edge_attr, batch, atom_num, W1a, b1a, W1b, b1b,
           root1, bias1, g1, be1, W2a, b2a, W2b, b2b, root2, bias2, g2, be2,
           W3a, b3a, W3b, b3b, root3, bias3, g3, be3, Wm, bm):
    f32 = jnp.float32
    src2 = edge_index[0].reshape(NW, NCHUNK, CHUNK)
    dst2 = edge_index[1].reshape(NW, NCHUNK, CHUNK)

    bf16 = jnp.bfloat16
    W2_1 = jnp.pad(W1b.reshape(K, 15, 64), ((0, 0), (0, 1), (0, 64)))
    W2_1 = W2_1.reshape(K * 16, 128)
    Bb1 = jnp.pad(b1b.reshape(15, 64), ((0, 1), (0, 64)))
    W2_2 = W2b.reshape(K * 64, 128)
    Bb2 = b2b.reshape(64, 128)
    W2_3 = W3b.reshape(K * 128, 256).astype(bf16)
    Bb3 = b3b.reshape(128, 256)

    cnt_row = jnp.zeros((1, 128), f32).at[0, CNTCOL].set(1.0)
    z128 = jnp.zeros((1, 128), f32)
    z256 = jnp.zeros((1, 256), f32)
    zN128 = jnp.zeros((N, 128), f32)

    xp1 = jnp.pad(x, ((0, 0), (0, 113)))
    scat = _sc_scatter_add(128)
    gat = _sc_gather(128)

    xg1 = gat(xp1, src2)
    (m1,) = _msg(xg1, edge_attr, W1a, b1a.reshape(1, K), W2_1, Bb1,
                 cnt_row, 128, 16)
    part1 = scat(m1, dst2, zN128)
    h1 = _finalize(part1, part1, x, root1, bias1.reshape(1, 64),
                   g1.reshape(1, 64), be1.reshape(1, 64), 64)

    h1p = jnp.pad(h1, ((0, 0), (0, 64)))
    xg2 = gat(h1p, src2)
    (m2,) = _msg(xg2, edge_attr, W2a, b2a.reshape(1, K), W2_2, Bb2,
                 z128, 128, 64)
    part2 = scat(m2, dst2, zN128)
    h2 = _finalize(part2, part1, h1, root2, bias2.reshape(1, 128),
                   g2.reshape(1, 128), be2.reshape(1, 128), 128)

    xg3 = gat(h2, src2)
    m3a, m3b = _msg(xg3, edge_attr, W3a, b3a.reshape(1, K), W2_3, Bb3,
                    z256, 256, 128)
    part3 = _sc_scatter_add2(128)(m3a, m3b, dst2, zN128)
    out, node_emb = _fin3_head(part3, part1, h2, root3, bias3.reshape(1, 256),
                               g3.reshape(1, 256), be3.reshape(1, 256),
                               batch.reshape(1, N), atom_num.reshape(G, 1),
                               Wm, bm.reshape(1, 200))
    return (out, node_emb)


# speedup vs baseline: 2.1065x; 1.1018x over previous
import functools

import jax
import jax.numpy as jnp
from jax import lax
from jax.experimental import pallas as pl
from jax.experimental.pallas import tpu as pltpu
from jax.experimental.pallas import tpu_sc as plsc

N = 3200
E = 12800
G = 128
K = 32

NC = 2
NS = 16
NW = NC * NS
EPW = E // NW
CHUNK = 80
NCHUNK = EPW // CHUNK
NPT = N // NS
CNTCOL = 64

_MESH = plsc.VectorSubcoreMesh(core_axis_name="c", subcore_axis_name="s")


def _sc_gather(D, dtype=jnp.float32):
    @functools.partial(
        pl.kernel,
        out_type=jax.ShapeDtypeStruct((E, D), dtype),
        mesh=_MESH,
        scratch_types=[
            pltpu.VMEM((NCHUNK, CHUNK), jnp.int32),
            pltpu.VMEM((EPW, D), dtype),
            pltpu.SemaphoreType.DMA,
        ],
    )
    def k(table_hbm, idx_hbm, out_hbm, idx_v, rows_v, sem):
        wid = lax.axis_index("s") * NC + lax.axis_index("c")
        pltpu.sync_copy(idx_hbm.at[wid], idx_v)
        copies = [
            pltpu.async_copy(
                table_hbm.at[idx_v.at[j]],
                rows_v.at[pl.ds(j * CHUNK, CHUNK)],
                sem,
            )
            for j in range(NCHUNK)
        ]
        for c in copies:
            c.wait()
        pltpu.sync_copy(rows_v, out_hbm.at[pl.ds(wid * EPW, EPW)])

    return k


def _sc_scatter_add(C):
    @functools.partial(
        pl.kernel,
        out_type=pltpu.HBM((NC, N, C), jnp.float32),
        mesh=_MESH,
        scratch_types=[
            pltpu.VMEM((NCHUNK, CHUNK), jnp.int32),
            pltpu.VMEM((EPW, C), jnp.float32),
            pltpu.VMEM_SHARED((N, C), jnp.float32),
        ],
    )
    def k(rows_hbm, idx_hbm, zeros_hbm, out_hbm, idx_v, rows_v, acc_sh):
        cid = lax.axis_index("c")
        sid = lax.axis_index("s")
        wid = sid * NC + cid
        pltpu.sync_copy(zeros_hbm.at[pl.ds(sid * NPT, NPT)],
                        acc_sh.at[pl.ds(sid * NPT, NPT)])
        pltpu.sync_copy(idx_hbm.at[wid], idx_v)
        pltpu.sync_copy(rows_hbm.at[pl.ds(wid * EPW, EPW)], rows_v)
        plsc.subcore_barrier()
        for j in range(NCHUNK):
            pltpu.sync_copy(rows_v.at[pl.ds(j * CHUNK, CHUNK)],
                            acc_sh.at[idx_v.at[j]], add=True)
        plsc.subcore_barrier()
        pltpu.sync_copy(acc_sh.at[pl.ds(sid * NPT, NPT)],
                        out_hbm.at[cid, pl.ds(sid * NPT, NPT)])

    return k


def _sc_scatter_add2(C):
    @functools.partial(
        pl.kernel,
        out_type=pltpu.HBM((NC, 2, N, C), jnp.float32),
        mesh=_MESH,
        scratch_types=[
            pltpu.VMEM((NCHUNK, CHUNK), jnp.int32),
            pltpu.VMEM((EPW, C), jnp.float32),
            pltpu.VMEM_SHARED((N, C), jnp.float32),
        ],
    )
    def k(rows_a, rows_b, idx_hbm, zeros_hbm, out_hbm, idx_v, rows_v, acc_sh):
        cid = lax.axis_index("c")
        sid = lax.axis_index("s")
        wid = sid * NC + cid
        pltpu.sync_copy(idx_hbm.at[wid], idx_v)
        for half, rows_hbm in enumerate((rows_a, rows_b)):
            pltpu.sync_copy(zeros_hbm.at[pl.ds(sid * NPT, NPT)],
                            acc_sh.at[pl.ds(sid * NPT, NPT)])
            pltpu.sync_copy(rows_hbm.at[pl.ds(wid * EPW, EPW)], rows_v)
            plsc.subcore_barrier()
            for j in range(NCHUNK):
                pltpu.sync_copy(rows_v.at[pl.ds(j * CHUNK, CHUNK)],
                                acc_sh.at[idx_v.at[j]], add=True)
            plsc.subcore_barrier()
            pltpu.sync_copy(acc_sh.at[pl.ds(sid * NPT, NPT)],
                            out_hbm.at[cid, half, pl.ds(sid * NPT, NPT)])

    return k


def _msg_body(xg_ref, ea_ref, wa_ref, ba_ref, w2_ref, bb_ref, ex_ref,
              *out_refs, D):
    h = jnp.maximum(ea_ref[...] @ wa_ref[...] + ba_ref[...], 0.0)
    xg = xg_ref[...][:, :D].astype(jnp.float32)
    acc = xg @ bb_ref[...] + ex_ref[...]
    if D == 128:
        xgb = xg.astype(jnp.bfloat16)
        hb = h.astype(jnp.bfloat16)
        y = jnp.concatenate(
            [xgb * hb[:, k:k + 1] for k in range(K)], axis=1)
        acc = acc + lax.dot(y, w2_ref[...],
                            preferred_element_type=jnp.float32)
    else:
        xgb = xg.astype(jnp.bfloat16)
        hb = h.astype(jnp.bfloat16)
        for k in range(K):
            acc = acc + lax.dot(xgb * hb[:, k:k + 1],
                                w2_ref[pl.ds(k * D, D), :],
                                preferred_element_type=jnp.float32)
    for i, o in enumerate(out_refs):
        o[...] = acc[:, i * 128:(i + 1) * 128]


def _msg(xg, ea, Wa, ba, W2, Bb, extra, Cp, D):
    EB = 1600
    nout = Cp // 128
    return pl.pallas_call(
        functools.partial(_msg_body, D=D),
        grid=(E // EB,),
        in_specs=[
            pl.BlockSpec((EB, 128), lambda i: (i, 0)),
            pl.BlockSpec((EB, 3), lambda i: (i, 0)),
            pl.BlockSpec((3, K), lambda i: (0, 0)),
            pl.BlockSpec((1, K), lambda i: (0, 0)),
            pl.BlockSpec((K * D, Cp), lambda i: (0, 0)),
            pl.BlockSpec((D, Cp), lambda i: (0, 0)),
            pl.BlockSpec((1, Cp), lambda i: (0, 0)),
        ],
        out_specs=[pl.BlockSpec((EB, 128), lambda i: (i, 0))] * nout,
        out_shape=[jax.ShapeDtypeStruct((E, 128), jnp.float32)] * nout,
    )(xg, ea, Wa, ba, W2, Bb, extra)


def _finalize_body(p_ref, c_ref, x_ref, root_ref, bias_ref, g_ref, b_ref,
                   out_ref):
    C = root_ref.shape[1]
    s = (p_ref[0] + p_ref[1])[:, :C]
    cnt = (c_ref[0] + c_ref[1])[:, CNTCOL:CNTCOL + 1]
    z = s / jnp.maximum(cnt, 1.0) + x_ref[...] @ root_ref[...] + bias_ref[...]
    mu = jnp.mean(z, axis=0, keepdims=True)
    var = jnp.mean((z - mu) ** 2, axis=0, keepdims=True)
    zn = g_ref[...] * (z - mu) * lax.rsqrt(var + 1e-5) + b_ref[...]
    out_ref[...] = jnp.maximum(zn, 0.0)


def _finalize(part, cntp, x_nodes, root, bias, gamma, beta, C):
    return pl.pallas_call(
        _finalize_body,
        out_shape=jax.ShapeDtypeStruct((N, C), jnp.float32),
    )(part, cntp, x_nodes, root, bias, gamma, beta)


def _fin3_head_body(p_ref, c_ref, x_ref, root_ref, bias_ref, g_ref, b3_ref,
                    batch_ref, atom_ref, wm_ref, bm_ref, out_ref, ne_ref):
    p = p_ref[...]
    s = jnp.concatenate([p[0, 0] + p[1, 0], p[0, 1] + p[1, 1]], axis=1)
    cnt = (c_ref[0] + c_ref[1])[:, CNTCOL:CNTCOL + 1]
    z = s / jnp.maximum(cnt, 1.0) + x_ref[...] @ root_ref[...] + bias_ref[...]
    mu = jnp.mean(z, axis=0, keepdims=True)
    var = jnp.mean((z - mu) ** 2, axis=0, keepdims=True)
    h = jnp.maximum(g_ref[...] * (z - mu) * lax.rsqrt(var + 1e-5)
                    + b3_ref[...], 0.0)
    b = batch_ref[...]
    gids = lax.broadcasted_iota(jnp.int32, (G, N), 0)
    member = (b == gids)
    onehot = member.astype(jnp.float32)
    sizes_i = jnp.sum(member.astype(jnp.int32), axis=1, keepdims=True)
    gsum = onehot @ h
    graph_emb = gsum / jnp.maximum(sizes_i.astype(jnp.float32), 1.0)
    cum = jnp.sum((b < gids).astype(jnp.int32), axis=1, keepdims=True)
    mod = jnp.minimum(atom_ref[...] + cum, N - 1)
    nids = lax.broadcasted_iota(jnp.int32, (G, N), 1)
    node_emb = (nids == mod).astype(jnp.float32) @ h
    hn = 0.5 * graph_emb + node_emb
    out_ref[...] = hn @ wm_ref[...] + bm_ref[...]
    ne_ref[...] = node_emb


def _fin3_head(part3, cntp, h2, root, bias, gamma, beta, batch2, atom2,
               Wm, bm2):
    return pl.pallas_call(
        _fin3_head_body,
        out_shape=(
            jax.ShapeDtypeStruct((G, 200), jnp.float32),
            jax.ShapeDtypeStruct((G, 256), jnp.float32),
        ),
    )(part3, cntp, h2, root, bias, gamma, beta, batch2, atom2, Wm, bm2)


def kernel(x, edge_index, edge_attr, batch, atom_num, W1a, b1a, W1b, b1b,
           root1, bias1, g1, be1, W2a, b2a, W2b, b2b, root2, bias2, g2, be2,
           W3a, b3a, W3b, b3b, root3, bias3, g3, be3, Wm, bm):
    f32 = jnp.float32
    src2 = edge_index[0].reshape(NW, NCHUNK, CHUNK)
    dst2 = edge_index[1].reshape(NW, NCHUNK, CHUNK)

    bf16 = jnp.bfloat16
    W2_1 = jnp.pad(W1b.reshape(K, 15, 64), ((0, 0), (0, 1), (0, 64)))
    W2_1 = W2_1.reshape(K * 16, 128).astype(bf16)
    Bb1 = jnp.pad(b1b.reshape(15, 64), ((0, 1), (0, 64)))
    W2_2 = W2b.reshape(K * 64, 128).astype(bf16)
    Bb2 = b2b.reshape(64, 128)
    W2_3 = W3b.reshape(K * 128, 256).astype(bf16)
    Bb3 = b3b.reshape(128, 256)

    cnt_row = jnp.zeros((1, 128), f32).at[0, CNTCOL].set(1.0)
    z128 = jnp.zeros((1, 128), f32)
    z256 = jnp.zeros((1, 256), f32)
    zN128 = jnp.zeros((N, 128), f32)

    xp1 = jnp.pad(x, ((0, 0), (0, 113)))
    scat = _sc_scatter_add(128)
    gat = _sc_gather(128)

    xg1 = gat(xp1, src2)
    (m1,) = _msg(xg1, edge_attr, W1a, b1a.reshape(1, K), W2_1, Bb1,
                 cnt_row, 128, 16)
    part1 = scat(m1, dst2, zN128)
    h1 = _finalize(part1, part1, x, root1, bias1.reshape(1, 64),
                   g1.reshape(1, 64), be1.reshape(1, 64), 64)

    h1p = jnp.pad(h1, ((0, 0), (0, 64)))
    xg2 = gat(h1p, src2)
    (m2,) = _msg(xg2, edge_attr, W2a, b2a.reshape(1, K), W2_2, Bb2,
                 z128, 128, 64)
    part2 = scat(m2, dst2, zN128)
    h2 = _finalize(part2, part1, h1, root2, bias2.reshape(1, 128),
                   g2.reshape(1, 128), be2.reshape(1, 128), 128)

    xg3 = gat(h2, src2)
    m3a, m3b = _msg(xg3, edge_attr, W3a, b3a.reshape(1, K), W2_3, Bb3,
                    z256, 256, 128)
    part3 = _sc_scatter_add2(128)(m3a, m3b, dst2, zN128)
    out, node_emb = _fin3_head(part3, part1, h2, root3, bias3.reshape(1, 256),
                               g3.reshape(1, 256), be3.reshape(1, 256),
                               batch.reshape(1, N), atom_num.reshape(G, 1),
                               Wm, bm.reshape(1, 200))
    return (out, node_emb)
